# Initial kernel scaffold; baseline (speedup 1.0000x reference)
import jax
import jax.numpy as jnp
from jax.experimental import pallas as pl


def kernel(user_indices, item_indices, anime_features, user_MF, item_MF, user_MLP, item_MLP, W_feat, b_feat, W1, b1, W2, b2, W3, b3, W4, b4, Wp, bp):
    raise NotImplementedError("write your pallas kernel here")


# speedup vs baseline: 2.2048x; 2.2048x over previous
import functools

import jax
import jax.numpy as jnp
from jax import lax
from jax.experimental import pallas as pl
from jax.experimental.pallas import tpu as pltpu
from jax.experimental.pallas import tpu_sc as plsc

_NC, _NS = 2, 16
_NW = _NC * _NS
_CAT = 16
_TBL = 128


def _collapse_body(W1, W2, W3, W4, Wp, W_feat, b_feat, b1, b2, b3, b4, bp,
                   cu_o, ci_o, caf_o, w16_o, c0_o):
    Wp_v = Wp[...]
    v4 = Wp_v[8:40]
    u3 = jnp.dot(W4[...], v4, preferred_element_type=jnp.float32)
    u2 = jnp.dot(W3[...], u3, preferred_element_type=jnp.float32)
    u1 = jnp.dot(W2[...], u2, preferred_element_type=jnp.float32)
    c = jnp.dot(W1[...], u1, preferred_element_type=jnp.float32)
    c3 = c[384:576]
    cu_o[...] = c[0:192]
    ci_o[...] = c[192:384]
    caf_o[...] = jnp.dot(W_feat[...], c3, preferred_element_type=jnp.float32)
    row = lax.broadcasted_iota(jnp.int32, (_TBL, 1), 0)
    wp_pad = jnp.concatenate(
        [Wp_v, jnp.zeros((_TBL - 40, 1), jnp.float32)], axis=0)
    w16_o[...] = jnp.where(row < 8, wp_pad,
                           jnp.where(row < 10, 1.0, 0.0))
    c0_o[...] = (jnp.dot(b_feat[...], c3, preferred_element_type=jnp.float32)
                 + jnp.dot(b1[...], u1, preferred_element_type=jnp.float32)
                 + jnp.dot(b2[...], u2, preferred_element_type=jnp.float32)
                 + jnp.dot(b3[...], u3, preferred_element_type=jnp.float32)
                 + jnp.dot(b4[...], v4, preferred_element_type=jnp.float32)
                 + bp[...])


def _collapse(W1, W2, W3, W4, Wp, W_feat, b_feat, b1, b2, b3, b4, bp):
    f32 = jnp.float32
    out_shape = (
        jax.ShapeDtypeStruct((192, 1), f32),
        jax.ShapeDtypeStruct((192, 1), f32),
        jax.ShapeDtypeStruct((8, 1), f32),
        jax.ShapeDtypeStruct((_TBL, 1), f32),
        jax.ShapeDtypeStruct((1, 1), f32),
    )
    return pl.pallas_call(_collapse_body, out_shape=out_shape)(
        W1, W2, W3, W4, Wp, W_feat, b_feat, b1, b2, b3, b4, bp)


_ROWS_BLK = 2000


def _project_body(u_mlp, i_mlp, u_mf, i_mf, cu, ci, ucat_o, icat_o):
    r = u_mf.shape[0]
    pu = jnp.dot(u_mlp[...], cu[...], preferred_element_type=jnp.float32)
    pi = jnp.dot(i_mlp[...], ci[...], preferred_element_type=jnp.float32)
    ones = jnp.ones((r, 1), jnp.float32)
    zeros = jnp.zeros((r, _TBL - 10), jnp.float32)
    ucat_o[...] = jnp.concatenate([u_mf[...], pu, ones, zeros], axis=1)
    icat_o[...] = jnp.concatenate([i_mf[...], ones, pi, zeros], axis=1)


def _project(user_MLP, item_MLP, user_MF, item_MF, cu, ci):
    n, d = user_MLP.shape
    r = _ROWS_BLK
    f32 = jnp.float32
    grid = (n // r,)
    out_shape = (
        jax.ShapeDtypeStruct((n, _TBL), f32),
        jax.ShapeDtypeStruct((n, _TBL), f32),
    )
    return pl.pallas_call(
        _project_body,
        grid=grid,
        in_specs=[
            pl.BlockSpec((r, d), lambda i: (i, 0)),
            pl.BlockSpec((r, d), lambda i: (i, 0)),
            pl.BlockSpec((r, 8), lambda i: (i, 0)),
            pl.BlockSpec((r, 8), lambda i: (i, 0)),
            pl.BlockSpec((d, 1), lambda i: (0, 0)),
            pl.BlockSpec((d, 1), lambda i: (0, 0)),
        ],
        out_specs=[
            pl.BlockSpec((r, _TBL), lambda i: (i, 0)),
            pl.BlockSpec((r, _TBL), lambda i: (i, 0)),
        ],
        out_shape=out_shape,
    )(user_MLP, item_MLP, user_MF, item_MF, cu, ci)


def _sc_gather(ucat, icat, uidx, iidx):
    b = uidx.shape[0]
    bpw = b // _NW
    chunk = bpw // 2
    f32 = jnp.float32
    mesh = plsc.VectorSubcoreMesh(core_axis_name="c", subcore_axis_name="s")
    out_type = (
        jax.ShapeDtypeStruct((b, 128), f32),
        jax.ShapeDtypeStruct((b, 128), f32),
    )

    @functools.partial(
        pl.kernel, mesh=mesh, out_type=out_type,
        scratch_types=[
            pltpu.VMEM((bpw,), jnp.int32),
            pltpu.VMEM((bpw,), jnp.int32),
            pltpu.VMEM((chunk, 128), f32),
            pltpu.VMEM((chunk, 128), f32),
            pltpu.SemaphoreType.DMA,
            pltpu.SemaphoreType.DMA,
        ],
    )
    def k(ucat_hbm, icat_hbm, uidx_hbm, iidx_hbm, ug_hbm, ig_hbm,
          idxu_v, idxi_v, rowsu_v, rowsi_v, semu, semi):
        wid = lax.axis_index("s") * _NC + lax.axis_index("c")
        base = wid * bpw
        pltpu.sync_copy(uidx_hbm.at[pl.ds(base, bpw)], idxu_v)
        pltpu.sync_copy(iidx_hbm.at[pl.ds(base, bpw)], idxi_v)
        cp_u = pltpu.async_copy(
            ucat_hbm.at[idxu_v.at[pl.ds(0, chunk)]], rowsu_v, semu)
        cp_i = pltpu.async_copy(
            icat_hbm.at[idxi_v.at[pl.ds(0, chunk)]], rowsi_v, semi)
        cp_u.wait()
        pltpu.sync_copy(rowsu_v, ug_hbm.at[pl.ds(base, chunk)])
        cp_u2 = pltpu.async_copy(
            ucat_hbm.at[idxu_v.at[pl.ds(chunk, chunk)]], rowsu_v, semu)
        cp_i.wait()
        pltpu.sync_copy(rowsi_v, ig_hbm.at[pl.ds(base, chunk)])
        cp_i2 = pltpu.async_copy(
            icat_hbm.at[idxi_v.at[pl.ds(chunk, chunk)]], rowsi_v, semi)
        cp_u2.wait()
        pltpu.sync_copy(rowsu_v, ug_hbm.at[pl.ds(base + chunk, chunk)])
        cp_i2.wait()
        pltpu.sync_copy(rowsi_v, ig_hbm.at[pl.ds(base + chunk, chunk)])

    return k(ucat, icat, uidx, iidx)


_COMB_BLK = 2048


def _combine_body(ug, ig, af, w16, caf, c0, o):
    prod = ug[...] * ig[...]
    pred = (jnp.dot(prod, w16[...], preferred_element_type=jnp.float32)
            + jnp.dot(af[...], caf[...], preferred_element_type=jnp.float32)
            + c0[...])
    o[...] = jax.nn.sigmoid(pred)


def _combine(ug, ig, af, w16, caf, c0):
    b = ug.shape[0]
    r = _COMB_BLK
    grid = (b // r,)
    return pl.pallas_call(
        _combine_body,
        grid=grid,
        in_specs=[
            pl.BlockSpec((r, _TBL), lambda i: (i, 0)),
            pl.BlockSpec((r, _TBL), lambda i: (i, 0)),
            pl.BlockSpec((r, 8), lambda i: (i, 0)),
            pl.BlockSpec((_TBL, 1), lambda i: (0, 0)),
            pl.BlockSpec((8, 1), lambda i: (0, 0)),
            pl.BlockSpec((1, 1), lambda i: (0, 0)),
        ],
        out_specs=pl.BlockSpec((r, 1), lambda i: (i, 0)),
        out_shape=jax.ShapeDtypeStruct((b, 1), jnp.float32),
    )(ug, ig, af, w16, caf, c0)


def kernel(user_indices, item_indices, anime_features, user_MF, item_MF,
           user_MLP, item_MLP, W_feat, b_feat, W1, b1, W2, b2, W3, b3, W4,
           b4, Wp, bp):
    cu, ci, caf, w16, c0 = _collapse(
        W1, W2, W3, W4, Wp, W_feat,
        b_feat.reshape(1, -1), b1.reshape(1, -1), b2.reshape(1, -1),
        b3.reshape(1, -1), b4.reshape(1, -1), bp.reshape(1, 1))
    ucat, icat = _project(user_MLP, item_MLP, user_MF, item_MF, cu, ci)
    ug, ig = _sc_gather(ucat, icat, user_indices, item_indices)
    return _combine(ug, ig, anime_features, w16, caf, c0)


# speedup vs baseline: 5.3500x; 2.4266x over previous
import functools

import jax
import jax.numpy as jnp
from jax import lax
from jax.experimental import pallas as pl
from jax.experimental.pallas import tpu as pltpu
from jax.experimental.pallas import tpu_sc as plsc

_NC, _NS = 2, 16
_NW = _NC * _NS
_CAT = 16
_TBL = 128
_F32 = jnp.float32


def _dn(a, b):
    return (((a,), (b,)), ((), ()))


def _dg(a, b, dn):
    return lax.dot_general(a, b, dn, preferred_element_type=_F32)


def _collapse_body(W1, W2, W3T, W4T, WpT, W_feat, b_feat, b1, b2, b3, b4, bp,
                   cuT_o, ciT_o, cafT_o, w16_o, c0_o):
    WpT_v = WpT[...]
    v4T = WpT_v[:, 8:40]
    u3T = _dg(v4T, W4T[...], _dn(1, 0))
    u2T = _dg(u3T, W3T[...], _dn(1, 0))
    u1T = _dg(u2T, W2[...], _dn(1, 1))
    cT = _dg(u1T, W1[...], _dn(1, 1))
    c3T = cT[:, 384:576]
    cuT_o[...] = cT[:, 0:192]
    ciT_o[...] = cT[:, 192:384]
    cafT_o[...] = _dg(c3T, W_feat[...], _dn(1, 1))
    row = lax.broadcasted_iota(jnp.int32, (_TBL, 40), 0)
    col = lax.broadcasted_iota(jnp.int32, (_TBL, 40), 1)
    sel = jnp.where((row == col) & (row < 8), 1.0, 0.0)
    w8 = _dg(sel, WpT_v, _dn(1, 1))
    r1 = lax.broadcasted_iota(jnp.int32, (_TBL, 1), 0)
    w16_o[...] = w8 + jnp.where((r1 >= 8) & (r1 < 10), 1.0, 0.0)
    c0_o[...] = (_dg(b_feat[...], c3T, _dn(1, 1))
                 + _dg(b1[...], u1T, _dn(1, 1))
                 + _dg(b2[...], u2T, _dn(1, 1))
                 + _dg(b3[...], u3T, _dn(1, 1))
                 + _dg(b4[...], v4T, _dn(1, 1))
                 + bp[...])


def _collapse(W1, W2, W3T, W4T, WpT, W_feat, b_feat, b1, b2, b3, b4, bp):
    out_shape = (
        jax.ShapeDtypeStruct((1, 192), _F32),
        jax.ShapeDtypeStruct((1, 192), _F32),
        jax.ShapeDtypeStruct((1, 8), _F32),
        jax.ShapeDtypeStruct((_TBL, 1), _F32),
        jax.ShapeDtypeStruct((1, 1), _F32),
    )
    return pl.pallas_call(_collapse_body, out_shape=out_shape)(
        W1, W2, W3T, W4T, WpT, W_feat, b_feat, b1, b2, b3, b4, bp)


_COLS_BLK = 2048


def _project_body(u_mlpT, i_mlpT, u_mfT, i_mfT, cuT, ciT, ucat_o, icat_o):
    c = u_mlpT.shape[1]
    pu = _dg(u_mlpT[...], cuT[...], _dn(0, 1))
    pi = _dg(i_mlpT[...], ciT[...], _dn(0, 1))
    r8 = lax.broadcasted_iota(jnp.int32, (8, 8), 0)
    c8 = lax.broadcasted_iota(jnp.int32, (8, 8), 1)
    eye8 = jnp.where(r8 == c8, 1.0, 0.0)
    umf = _dg(u_mfT[...], eye8, _dn(0, 0))
    imf = _dg(i_mfT[...], eye8, _dn(0, 0))
    ones = jnp.ones((c, 1), _F32)
    zeros = jnp.zeros((c, _TBL - 10), _F32)
    ucat_o[...] = jnp.concatenate([umf, pu, ones, zeros], axis=1)
    icat_o[...] = jnp.concatenate([imf, ones, pi, zeros], axis=1)


def _project(user_MLPT, item_MLPT, user_MFT, item_MFT, cuT, ciT):
    d, n = user_MLPT.shape
    c = _COLS_BLK
    grid = (pl.cdiv(n, c),)
    out_shape = (
        jax.ShapeDtypeStruct((n, _TBL), _F32),
        jax.ShapeDtypeStruct((n, _TBL), _F32),
    )
    return pl.pallas_call(
        _project_body,
        grid=grid,
        in_specs=[
            pl.BlockSpec((d, c), lambda i: (0, i)),
            pl.BlockSpec((d, c), lambda i: (0, i)),
            pl.BlockSpec((8, c), lambda i: (0, i)),
            pl.BlockSpec((8, c), lambda i: (0, i)),
            pl.BlockSpec((1, 192), lambda i: (0, 0)),
            pl.BlockSpec((1, 192), lambda i: (0, 0)),
        ],
        out_specs=[
            pl.BlockSpec((c, _TBL), lambda i: (i, 0)),
            pl.BlockSpec((c, _TBL), lambda i: (i, 0)),
        ],
        out_shape=out_shape,
    )(user_MLPT, item_MLPT, user_MFT, item_MFT, cuT, ciT)


def _sc_gather(ucat, icat, uidx, iidx):
    b = uidx.shape[0]
    bpw = b // _NW
    chunk = bpw // 2
    mesh = plsc.VectorSubcoreMesh(core_axis_name="c", subcore_axis_name="s")
    out_type = (
        jax.ShapeDtypeStruct((b, _TBL), _F32),
        jax.ShapeDtypeStruct((b, _TBL), _F32),
    )

    @functools.partial(
        pl.kernel, mesh=mesh, out_type=out_type,
        scratch_types=[
            pltpu.VMEM((bpw,), jnp.int32),
            pltpu.VMEM((bpw,), jnp.int32),
            pltpu.VMEM((chunk, _TBL), _F32),
            pltpu.VMEM((chunk, _TBL), _F32),
            pltpu.SemaphoreType.DMA,
            pltpu.SemaphoreType.DMA,
        ],
    )
    def k(ucat_hbm, icat_hbm, uidx_hbm, iidx_hbm, ug_hbm, ig_hbm,
          idxu_v, idxi_v, rowsu_v, rowsi_v, semu, semi):
        wid = lax.axis_index("s") * _NC + lax.axis_index("c")
        base = wid * bpw
        pltpu.sync_copy(uidx_hbm.at[pl.ds(base, bpw)], idxu_v)
        pltpu.sync_copy(iidx_hbm.at[pl.ds(base, bpw)], idxi_v)
        cp_u = pltpu.async_copy(
            ucat_hbm.at[idxu_v.at[pl.ds(0, chunk)]], rowsu_v, semu)
        cp_i = pltpu.async_copy(
            icat_hbm.at[idxi_v.at[pl.ds(0, chunk)]], rowsi_v, semi)
        cp_u.wait()
        pltpu.sync_copy(rowsu_v, ug_hbm.at[pl.ds(base, chunk)])
        cp_u2 = pltpu.async_copy(
            ucat_hbm.at[idxu_v.at[pl.ds(chunk, chunk)]], rowsu_v, semu)
        cp_i.wait()
        pltpu.sync_copy(rowsi_v, ig_hbm.at[pl.ds(base, chunk)])
        cp_i2 = pltpu.async_copy(
            icat_hbm.at[idxi_v.at[pl.ds(chunk, chunk)]], rowsi_v, semi)
        cp_u2.wait()
        pltpu.sync_copy(rowsu_v, ug_hbm.at[pl.ds(base + chunk, chunk)])
        cp_i2.wait()
        pltpu.sync_copy(rowsi_v, ig_hbm.at[pl.ds(base + chunk, chunk)])

    return k(ucat, icat, uidx, iidx)


_COMB_BLK = 2048


def _combine_body(ug, ig, afT, w16, cafT, c0, o):
    prod = ug[...] * ig[...]
    predT = _dg(w16[...], prod, _dn(0, 1))
    featT = _dg(cafT[...], afT[...], _dn(1, 0))
    o[...] = jax.nn.sigmoid(predT + featT + c0[...])


def _combine(ug, ig, afT, w16, cafT, c0):
    b = ug.shape[0]
    c = _COMB_BLK
    grid = (b // c,)
    return pl.pallas_call(
        _combine_body,
        grid=grid,
        in_specs=[
            pl.BlockSpec((c, _TBL), lambda i: (i, 0)),
            pl.BlockSpec((c, _TBL), lambda i: (i, 0)),
            pl.BlockSpec((8, c), lambda i: (0, i)),
            pl.BlockSpec((_TBL, 1), lambda i: (0, 0)),
            pl.BlockSpec((1, 8), lambda i: (0, 0)),
            pl.BlockSpec((1, 1), lambda i: (0, 0)),
        ],
        out_specs=pl.BlockSpec((1, c), lambda i: (0, i)),
        out_shape=jax.ShapeDtypeStruct((1, b), _F32),
    )(ug, ig, afT, w16, cafT, c0)


def kernel(user_indices, item_indices, anime_features, user_MF, item_MF,
           user_MLP, item_MLP, W_feat, b_feat, W1, b1, W2, b2, W3, b3, W4,
           b4, Wp, bp):
    cuT, ciT, cafT, w16, c0 = _collapse(
        W1, W2, W3.T, W4.T, Wp.T, W_feat,
        b_feat.reshape(1, -1), b1.reshape(1, -1), b2.reshape(1, -1),
        b3.reshape(1, -1), b4.reshape(1, -1), bp.reshape(1, 1))
    ucat, icat = _project(
        user_MLP.T, item_MLP.T, user_MF.T, item_MF.T, cuT, ciT)
    ug, ig = _sc_gather(ucat, icat, user_indices, item_indices)
    outT = _combine(ug, ig, anime_features.T, w16, cafT, c0)
    return outT.T


# speedup vs baseline: 6.6781x; 1.2482x over previous
import functools

import jax
import jax.numpy as jnp
from jax import lax
from jax.experimental import pallas as pl
from jax.experimental.pallas import tpu as pltpu
from jax.experimental.pallas import tpu_sc as plsc

_NC, _NS = 2, 16
_NW = _NC * _NS
_CAT = 16
_TBL = 128
_PACK = _TBL // _CAT
_F32 = jnp.float32


def _dn(a, b):
    return (((a,), (b,)), ((), ()))


def _dg(a, b, dn):
    return lax.dot_general(a, b, dn, preferred_element_type=_F32)


def _collapse_body(W1, W2, W3T, W4T, WpT, W_feat, b_feat, b1, b2, b3, b4, bp,
                   cuT_o, ciT_o, cafT_o, w16_o, c0_o):
    WpT_v = WpT[...]
    v4T = WpT_v[:, 8:40]
    u3T = _dg(v4T, W4T[...], _dn(1, 0))
    u2T = _dg(u3T, W3T[...], _dn(1, 0))
    u1T = _dg(u2T, W2[...], _dn(1, 1))
    cT = _dg(u1T, W1[...], _dn(1, 1))
    c3T = cT[:, 384:576]
    cuT_o[...] = cT[:, 0:192]
    ciT_o[...] = cT[:, 192:384]
    cafT_o[...] = _dg(c3T, W_feat[...], _dn(1, 1))
    row = lax.broadcasted_iota(jnp.int32, (_CAT, 40), 0)
    col = lax.broadcasted_iota(jnp.int32, (_CAT, 40), 1)
    sel = jnp.where((row == col) & (row < 8), 1.0, 0.0)
    w8 = _dg(sel, WpT_v, _dn(1, 1))
    r1 = lax.broadcasted_iota(jnp.int32, (_CAT, 1), 0)
    w16_o[...] = w8 + jnp.where((r1 >= 8) & (r1 < 10), 1.0, 0.0)
    c0_o[...] = (_dg(b_feat[...], c3T, _dn(1, 1))
                 + _dg(b1[...], u1T, _dn(1, 1))
                 + _dg(b2[...], u2T, _dn(1, 1))
                 + _dg(b3[...], u3T, _dn(1, 1))
                 + _dg(b4[...], v4T, _dn(1, 1))
                 + bp[...])


def _collapse(W1, W2, W3T, W4T, WpT, W_feat, b_feat, b1, b2, b3, b4, bp):
    out_shape = (
        jax.ShapeDtypeStruct((1, 192), _F32),
        jax.ShapeDtypeStruct((1, 192), _F32),
        jax.ShapeDtypeStruct((1, 8), _F32),
        jax.ShapeDtypeStruct((_CAT, 1), _F32),
        jax.ShapeDtypeStruct((1, 1), _F32),
    )
    return pl.pallas_call(_collapse_body, out_shape=out_shape)(
        W1, W2, W3T, W4T, WpT, W_feat, b_feat, b1, b2, b3, b4, bp)


_COLS_BLK = 4096


def _project_body(u_mlpT, i_mlpT, u_mfT, i_mfT, cuT, ciT, cat_o):
    c = u_mlpT.shape[1]
    pu = _dg(u_mlpT[...], cuT[...], _dn(0, 1))
    pi = _dg(i_mlpT[...], ciT[...], _dn(0, 1))
    r8 = lax.broadcasted_iota(jnp.int32, (8, 8), 0)
    c8 = lax.broadcasted_iota(jnp.int32, (8, 8), 1)
    eye8 = jnp.where(r8 == c8, 1.0, 0.0)
    umf = _dg(u_mfT[...], eye8, _dn(0, 0))
    imf = _dg(i_mfT[...], eye8, _dn(0, 0))
    ones = jnp.ones((c, 1), _F32)
    z6 = jnp.zeros((c, 6), _F32)
    ztail = jnp.zeros((c, _TBL - 2 * _CAT), _F32)
    cat_o[...] = jnp.concatenate(
        [umf, pu, ones, z6, imf, ones, pi, z6, ztail], axis=1)


def _project(user_MLPT, item_MLPT, user_MFT, item_MFT, cuT, ciT):
    d, n = user_MLPT.shape
    c = _COLS_BLK
    grid = (pl.cdiv(n, c),)
    out_shape = jax.ShapeDtypeStruct((n, _TBL), _F32)
    return pl.pallas_call(
        _project_body,
        grid=grid,
        compiler_params=pltpu.CompilerParams(
            fuse_transposed_lhs_in_matmul=True),
        in_specs=[
            pl.BlockSpec((d, c), lambda i: (0, i)),
            pl.BlockSpec((d, c), lambda i: (0, i)),
            pl.BlockSpec((8, c), lambda i: (0, i)),
            pl.BlockSpec((8, c), lambda i: (0, i)),
            pl.BlockSpec((1, 192), lambda i: (0, 0)),
            pl.BlockSpec((1, 192), lambda i: (0, 0)),
        ],
        out_specs=pl.BlockSpec((c, _TBL), lambda i: (i, 0)),
        out_shape=out_shape,
    )(user_MLPT, item_MLPT, user_MFT, item_MFT, cuT, ciT)


def _sc_gather(cat, uidx, iidx):
    b = uidx.shape[0]
    bpw = b // _NW
    chunk = bpw // 2
    mesh = plsc.VectorSubcoreMesh(core_axis_name="c", subcore_axis_name="s")
    out_type = (
        jax.ShapeDtypeStruct((b, _TBL), _F32),
        jax.ShapeDtypeStruct((b, _TBL), _F32),
    )

    @functools.partial(
        pl.kernel, mesh=mesh, out_type=out_type,
        scratch_types=[
            pltpu.VMEM((bpw,), jnp.int32),
            pltpu.VMEM((bpw,), jnp.int32),
            pltpu.VMEM((chunk, _TBL), _F32),
            pltpu.VMEM((chunk, _TBL), _F32),
            pltpu.SemaphoreType.DMA,
            pltpu.SemaphoreType.DMA,
        ],
    )
    def k(cat_hbm, uidx_hbm, iidx_hbm, ug_hbm, ig_hbm,
          idxu_v, idxi_v, rowsu_v, rowsi_v, semu, semi):
        wid = lax.axis_index("s") * _NC + lax.axis_index("c")
        base = wid * bpw
        pltpu.sync_copy(uidx_hbm.at[pl.ds(base, bpw)], idxu_v)
        pltpu.sync_copy(iidx_hbm.at[pl.ds(base, bpw)], idxi_v)
        cp_u = pltpu.async_copy(
            cat_hbm.at[idxu_v.at[pl.ds(0, chunk)]], rowsu_v, semu)
        cp_i = pltpu.async_copy(
            cat_hbm.at[idxi_v.at[pl.ds(0, chunk)]], rowsi_v, semi)
        cp_u.wait()
        pltpu.sync_copy(rowsu_v, ug_hbm.at[pl.ds(base, chunk)])
        cp_u2 = pltpu.async_copy(
            cat_hbm.at[idxu_v.at[pl.ds(chunk, chunk)]], rowsu_v, semu)
        cp_i.wait()
        pltpu.sync_copy(rowsi_v, ig_hbm.at[pl.ds(base, chunk)])
        cp_i2 = pltpu.async_copy(
            cat_hbm.at[idxi_v.at[pl.ds(chunk, chunk)]], rowsi_v, semi)
        cp_u2.wait()
        pltpu.sync_copy(rowsu_v, ug_hbm.at[pl.ds(base + chunk, chunk)])
        cp_i2.wait()
        pltpu.sync_copy(rowsi_v, ig_hbm.at[pl.ds(base + chunk, chunk)])

    return k(cat, uidx, iidx)


_COMB_BLK = 2048


def _combine_body(ug, ig, afT, w16, cafT, c0, o):
    ue = ug[...][:, 0:_CAT]
    ie = ig[...][:, _CAT:2 * _CAT]
    prod = ue * ie
    predT = _dg(w16[...], prod, _dn(0, 1))
    featT = _dg(cafT[...], afT[...], _dn(1, 0))
    o[...] = jax.nn.sigmoid(predT + featT + c0[...])


def _combine(ug, ig, afT, w16, cafT, c0):
    b = ug.shape[0]
    c = _COMB_BLK
    grid = (b // c,)
    return pl.pallas_call(
        _combine_body,
        grid=grid,
        in_specs=[
            pl.BlockSpec((c, _TBL), lambda i: (i, 0)),
            pl.BlockSpec((c, _TBL), lambda i: (i, 0)),
            pl.BlockSpec((8, c), lambda i: (0, i)),
            pl.BlockSpec((_CAT, 1), lambda i: (0, 0)),
            pl.BlockSpec((1, 8), lambda i: (0, 0)),
            pl.BlockSpec((1, 1), lambda i: (0, 0)),
        ],
        out_specs=pl.BlockSpec((1, c), lambda i: (0, i)),
        out_shape=jax.ShapeDtypeStruct((1, b), _F32),
    )(ug, ig, afT, w16, cafT, c0)


def kernel(user_indices, item_indices, anime_features, user_MF, item_MF,
           user_MLP, item_MLP, W_feat, b_feat, W1, b1, W2, b2, W3, b3, W4,
           b4, Wp, bp):
    cuT, ciT, cafT, w16, c0 = _collapse(
        W1, W2, W3.T, W4.T, Wp.T, W_feat,
        b_feat.reshape(1, -1), b1.reshape(1, -1), b2.reshape(1, -1),
        b3.reshape(1, -1), b4.reshape(1, -1), bp.reshape(1, 1))
    cat = _project(
        user_MLP.T, item_MLP.T, user_MF.T, item_MF.T, cuT, ciT)
    ug, ig = _sc_gather(cat, user_indices, item_indices)
    outT = _combine(ug, ig, anime_features.T, w16, cafT, c0)
    return outT.T


# speedup vs baseline: 6.9112x; 1.0349x over previous
import functools

import jax
import jax.numpy as jnp
from jax import lax
from jax.experimental import pallas as pl
from jax.experimental.pallas import tpu as pltpu
from jax.experimental.pallas import tpu_sc as plsc

_NC, _NS = 2, 16
_NW = _NC * _NS
_CAT = 16
_TBL = 128
_PACK = _TBL // _CAT
_F32 = jnp.float32


def _dn(a, b):
    return (((a,), (b,)), ((), ()))


def _dg(a, b, dn):
    return lax.dot_general(a, b, dn, preferred_element_type=_F32)


def _collapse_body(W1, W2, W3T, W4T, WpT, W_feat, b_feat, b1, b2, b3, b4, bp,
                   cuT_o, ciT_o, cafT_o, w16_o, c0_o):
    WpT_v = WpT[...]
    v4T = WpT_v[:, 8:40]
    u3T = _dg(v4T, W4T[...], _dn(1, 0))
    u2T = _dg(u3T, W3T[...], _dn(1, 0))
    u1T = _dg(u2T, W2[...], _dn(1, 1))
    cT = _dg(u1T, W1[...], _dn(1, 1))
    c3T = cT[:, 384:576]
    cuT_o[...] = cT[:, 0:192]
    ciT_o[...] = cT[:, 192:384]
    cafT_o[...] = _dg(c3T, W_feat[...], _dn(1, 1))
    row = lax.broadcasted_iota(jnp.int32, (_CAT, 40), 0)
    col = lax.broadcasted_iota(jnp.int32, (_CAT, 40), 1)
    sel = jnp.where((row == col) & (row < 8), 1.0, 0.0)
    w8 = _dg(sel, WpT_v, _dn(1, 1))
    r1 = lax.broadcasted_iota(jnp.int32, (_CAT, 1), 0)
    w16_o[...] = w8 + jnp.where((r1 >= 8) & (r1 < 10), 1.0, 0.0)
    c0_o[...] = (_dg(b_feat[...], c3T, _dn(1, 1))
                 + _dg(b1[...], u1T, _dn(1, 1))
                 + _dg(b2[...], u2T, _dn(1, 1))
                 + _dg(b3[...], u3T, _dn(1, 1))
                 + _dg(b4[...], v4T, _dn(1, 1))
                 + bp[...])


def _collapse(W1, W2, W3T, W4T, WpT, W_feat, b_feat, b1, b2, b3, b4, bp):
    out_shape = (
        jax.ShapeDtypeStruct((1, 192), _F32),
        jax.ShapeDtypeStruct((1, 192), _F32),
        jax.ShapeDtypeStruct((1, 8), _F32),
        jax.ShapeDtypeStruct((_CAT, 1), _F32),
        jax.ShapeDtypeStruct((1, 1), _F32),
    )
    return pl.pallas_call(_collapse_body, out_shape=out_shape)(
        W1, W2, W3T, W4T, WpT, W_feat, b_feat, b1, b2, b3, b4, bp)


_COLS_BLK = 4096


def _project_body(u_mlpT, i_mlpT, u_mfT, i_mfT, cuT, ciT, cat_o):
    c = u_mlpT.shape[1]
    pu = _dg(u_mlpT[...], cuT[...], _dn(0, 1))
    pi = _dg(i_mlpT[...], ciT[...], _dn(0, 1))
    r8 = lax.broadcasted_iota(jnp.int32, (8, 8), 0)
    c8 = lax.broadcasted_iota(jnp.int32, (8, 8), 1)
    eye8 = jnp.where(r8 == c8, 1.0, 0.0)
    umf = _dg(u_mfT[...], eye8, _dn(0, 0))
    imf = _dg(i_mfT[...], eye8, _dn(0, 0))
    ones = jnp.ones((c, 1), _F32)
    z6 = jnp.zeros((c, 6), _F32)
    ztail = jnp.zeros((c, _TBL - 2 * _CAT), _F32)
    cat_o[...] = jnp.concatenate(
        [umf, pu, ones, z6, imf, ones, pi, z6, ztail], axis=1)


def _project(user_MLPT, item_MLPT, user_MFT, item_MFT, cuT, ciT):
    d, n = user_MLPT.shape
    c = _COLS_BLK
    grid = (pl.cdiv(n, c),)
    out_shape = jax.ShapeDtypeStruct((n, _TBL), _F32)
    return pl.pallas_call(
        _project_body,
        grid=grid,
        compiler_params=pltpu.CompilerParams(
            fuse_transposed_lhs_in_matmul=True),
        in_specs=[
            pl.BlockSpec((d, c), lambda i: (0, i)),
            pl.BlockSpec((d, c), lambda i: (0, i)),
            pl.BlockSpec((8, c), lambda i: (0, i)),
            pl.BlockSpec((8, c), lambda i: (0, i)),
            pl.BlockSpec((1, 192), lambda i: (0, 0)),
            pl.BlockSpec((1, 192), lambda i: (0, 0)),
        ],
        out_specs=pl.BlockSpec((c, _TBL), lambda i: (i, 0)),
        out_shape=out_shape,
    )(user_MLPT, item_MLPT, user_MFT, item_MFT, cuT, ciT)


def _sc_gather(cat, uidx, iidx):
    b = uidx.shape[0]
    bpw = b // _NW
    chunk = bpw // 2
    mesh = plsc.VectorSubcoreMesh(core_axis_name="c", subcore_axis_name="s")
    out_type = jax.ShapeDtypeStruct((b, _TBL), _F32)

    @functools.partial(
        pl.kernel, mesh=mesh, out_type=out_type,
        scratch_types=[
            pltpu.VMEM((bpw,), jnp.int32),
            pltpu.VMEM((bpw,), jnp.int32),
            pltpu.VMEM((chunk, _TBL), _F32),
            pltpu.VMEM((chunk, _TBL), _F32),
            pltpu.SemaphoreType.DMA,
            pltpu.SemaphoreType.DMA,
        ],
    )
    def k(cat_hbm, uidx_hbm, iidx_hbm, ugi_hbm,
          idxu_v, idxi_v, rowsu_v, rowsi_v, semu, semi):
        wid = lax.axis_index("s") * _NC + lax.axis_index("c")
        base = wid * bpw
        pltpu.sync_copy(uidx_hbm.at[pl.ds(base, bpw)], idxu_v)
        pltpu.sync_copy(iidx_hbm.at[pl.ds(base, bpw)], idxi_v)
        cp_u = pltpu.async_copy(
            cat_hbm.at[idxu_v.at[pl.ds(0, chunk)]], rowsu_v, semu)
        cp_i = pltpu.async_copy(
            cat_hbm.at[idxi_v.at[pl.ds(0, chunk)]], rowsi_v, semi)
        def merge():
            @pl.loop(0, chunk, step=8)
            def _(r):
                for j in range(8):
                    sl = pl.ds(_CAT, _CAT)
                    rowsu_v[r + j, sl] = rowsi_v[r + j, sl]

        cp_u.wait()
        cp_i.wait()
        merge()
        cp_i2 = pltpu.async_copy(
            cat_hbm.at[idxi_v.at[pl.ds(chunk, chunk)]], rowsi_v, semi)
        pltpu.sync_copy(rowsu_v, ugi_hbm.at[pl.ds(base, chunk)])
        cp_u2 = pltpu.async_copy(
            cat_hbm.at[idxu_v.at[pl.ds(chunk, chunk)]], rowsu_v, semu)
        cp_u2.wait()
        cp_i2.wait()
        merge()
        pltpu.sync_copy(rowsu_v, ugi_hbm.at[pl.ds(base + chunk, chunk)])

    return k(cat, uidx, iidx)


_COMB_BLK = 2048


def _combine_body(ugi, afT, w16, cafT, c0, o):
    ue = ugi[...][:, 0:_CAT]
    ie = ugi[...][:, _CAT:2 * _CAT]
    prod = ue * ie
    predT = _dg(w16[...], prod, _dn(0, 1))
    featT = _dg(cafT[...], afT[...], _dn(1, 0))
    o[...] = jax.nn.sigmoid(predT + featT + c0[...])


def _combine(ugi, afT, w16, cafT, c0):
    b = ugi.shape[0]
    c = _COMB_BLK
    grid = (b // c,)
    return pl.pallas_call(
        _combine_body,
        grid=grid,
        in_specs=[
            pl.BlockSpec((c, _TBL), lambda i: (i, 0)),
            pl.BlockSpec((8, c), lambda i: (0, i)),
            pl.BlockSpec((_CAT, 1), lambda i: (0, 0)),
            pl.BlockSpec((1, 8), lambda i: (0, 0)),
            pl.BlockSpec((1, 1), lambda i: (0, 0)),
        ],
        out_specs=pl.BlockSpec((1, c), lambda i: (0, i)),
        out_shape=jax.ShapeDtypeStruct((1, b), _F32),
    )(ugi, afT, w16, cafT, c0)


def kernel(user_indices, item_indices, anime_features, user_MF, item_MF,
           user_MLP, item_MLP, W_feat, b_feat, W1, b1, W2, b2, W3, b3, W4,
           b4, Wp, bp):
    cuT, ciT, cafT, w16, c0 = _collapse(
        W1, W2, W3.T, W4.T, Wp.T, W_feat,
        b_feat.reshape(1, -1), b1.reshape(1, -1), b2.reshape(1, -1),
        b3.reshape(1, -1), b4.reshape(1, -1), bp.reshape(1, 1))
    cat = _project(
        user_MLP.T, item_MLP.T, user_MF.T, item_MF.T, cuT, ciT)
    ugi = _sc_gather(cat, user_indices, item_indices)
    outT = _combine(ugi, anime_features.T, w16, cafT, c0)
    return outT.T


# speedup vs baseline: 7.0257x; 1.0166x over previous
import functools

import jax
import jax.numpy as jnp
from jax import lax
from jax.experimental import pallas as pl
from jax.experimental.pallas import tpu as pltpu
from jax.experimental.pallas import tpu_sc as plsc

_NC, _NS = 2, 16
_NW = _NC * _NS
_CAT = 16
_TBL = 128
_PACK = _TBL // _CAT
_F32 = jnp.float32


def _dn(a, b):
    return (((a,), (b,)), ((), ()))


def _dg(a, b, dn):
    return lax.dot_general(a, b, dn, preferred_element_type=_F32)


def _collapse_body(W1, W2, W3T, W4T, WpT, W_feat, b_feat, b1, b2, b3, b4, bp,
                   cuT_o, ciT_o, cafT_o, w16_o, c0_o):
    WpT_v = WpT[...]
    v4T = WpT_v[:, 8:40]
    u3T = _dg(v4T, W4T[...], _dn(1, 0))
    u2T = _dg(u3T, W3T[...], _dn(1, 0))
    u1T = _dg(u2T, W2[...], _dn(1, 1))
    cT = _dg(u1T, W1[...], _dn(1, 1))
    c3T = cT[:, 384:576]
    cuT_o[...] = cT[:, 0:192]
    ciT_o[...] = cT[:, 192:384]
    cafT_o[...] = _dg(c3T, W_feat[...], _dn(1, 1))
    row = lax.broadcasted_iota(jnp.int32, (_CAT, 40), 0)
    col = lax.broadcasted_iota(jnp.int32, (_CAT, 40), 1)
    sel = jnp.where((row == col) & (row < 8), 1.0, 0.0)
    w8 = _dg(sel, WpT_v, _dn(1, 1))
    r1 = lax.broadcasted_iota(jnp.int32, (_CAT, 1), 0)
    w16_o[...] = w8 + jnp.where((r1 >= 8) & (r1 < 10), 1.0, 0.0)
    c0_o[...] = (_dg(b_feat[...], c3T, _dn(1, 1))
                 + _dg(b1[...], u1T, _dn(1, 1))
                 + _dg(b2[...], u2T, _dn(1, 1))
                 + _dg(b3[...], u3T, _dn(1, 1))
                 + _dg(b4[...], v4T, _dn(1, 1))
                 + bp[...])


def _collapse(W1, W2, W3T, W4T, WpT, W_feat, b_feat, b1, b2, b3, b4, bp):
    out_shape = (
        jax.ShapeDtypeStruct((1, 192), _F32),
        jax.ShapeDtypeStruct((1, 192), _F32),
        jax.ShapeDtypeStruct((1, 8), _F32),
        jax.ShapeDtypeStruct((_CAT, 1), _F32),
        jax.ShapeDtypeStruct((1, 1), _F32),
    )
    return pl.pallas_call(_collapse_body, out_shape=out_shape)(
        W1, W2, W3T, W4T, WpT, W_feat, b_feat, b1, b2, b3, b4, bp)


_COLS_BLK = 6144


def _project_body(u_mlpT, i_mlpT, u_mfT, i_mfT, cuT, ciT, cat_o):
    c = u_mlpT.shape[1]
    pu = _dg(u_mlpT[...], cuT[...], _dn(0, 1))
    pi = _dg(i_mlpT[...], ciT[...], _dn(0, 1))
    r8 = lax.broadcasted_iota(jnp.int32, (8, 8), 0)
    c8 = lax.broadcasted_iota(jnp.int32, (8, 8), 1)
    eye8 = jnp.where(r8 == c8, 1.0, 0.0)
    umf = _dg(u_mfT[...], eye8, _dn(0, 0))
    imf = _dg(i_mfT[...], eye8, _dn(0, 0))
    ones = jnp.ones((c, 1), _F32)
    z6 = jnp.zeros((c, 6), _F32)
    ztail = jnp.zeros((c, _TBL - 2 * _CAT), _F32)
    cat_o[...] = jnp.concatenate(
        [umf, pu, ones, z6, imf, ones, pi, z6, ztail], axis=1)


def _project(user_MLPT, item_MLPT, user_MFT, item_MFT, cuT, ciT):
    d, n = user_MLPT.shape
    c = _COLS_BLK
    grid = (pl.cdiv(n, c),)
    out_shape = jax.ShapeDtypeStruct((n, _TBL), _F32)
    return pl.pallas_call(
        _project_body,
        grid=grid,
        compiler_params=pltpu.CompilerParams(
            fuse_transposed_lhs_in_matmul=True),
        in_specs=[
            pl.BlockSpec((d, c), lambda i: (0, i)),
            pl.BlockSpec((d, c), lambda i: (0, i)),
            pl.BlockSpec((8, c), lambda i: (0, i)),
            pl.BlockSpec((8, c), lambda i: (0, i)),
            pl.BlockSpec((1, 192), lambda i: (0, 0)),
            pl.BlockSpec((1, 192), lambda i: (0, 0)),
        ],
        out_specs=pl.BlockSpec((c, _TBL), lambda i: (i, 0)),
        out_shape=out_shape,
    )(user_MLPT, item_MLPT, user_MFT, item_MFT, cuT, ciT)


def _sc_gather(cat, uidx, iidx):
    b = uidx.shape[0]
    bpw = b // _NW
    chunk = bpw // 2
    mesh = plsc.VectorSubcoreMesh(core_axis_name="c", subcore_axis_name="s")
    out_type = jax.ShapeDtypeStruct((b, _TBL), _F32)

    @functools.partial(
        pl.kernel, mesh=mesh, out_type=out_type,
        scratch_types=[
            pltpu.VMEM((bpw,), jnp.int32),
            pltpu.VMEM((bpw,), jnp.int32),
            pltpu.VMEM((chunk, _TBL), _F32),
            pltpu.VMEM((chunk, _TBL), _F32),
            pltpu.SemaphoreType.DMA,
            pltpu.SemaphoreType.DMA,
        ],
    )
    def k(cat_hbm, uidx_hbm, iidx_hbm, ugi_hbm,
          idxu_v, idxi_v, rowsu_v, rowsi_v, semu, semi):
        wid = lax.axis_index("s") * _NC + lax.axis_index("c")
        base = wid * bpw
        pltpu.sync_copy(uidx_hbm.at[pl.ds(base, bpw)], idxu_v)
        pltpu.sync_copy(iidx_hbm.at[pl.ds(base, bpw)], idxi_v)
        cp_u = pltpu.async_copy(
            cat_hbm.at[idxu_v.at[pl.ds(0, chunk)]], rowsu_v, semu)
        cp_i = pltpu.async_copy(
            cat_hbm.at[idxi_v.at[pl.ds(0, chunk)]], rowsi_v, semi)
        def merge():
            @pl.loop(0, chunk, step=8)
            def _(r):
                for j in range(8):
                    sl = pl.ds(_CAT, _CAT)
                    rowsu_v[r + j, sl] = rowsi_v[r + j, sl]

        cp_u.wait()
        cp_i.wait()
        merge()
        cp_i2 = pltpu.async_copy(
            cat_hbm.at[idxi_v.at[pl.ds(chunk, chunk)]], rowsi_v, semi)
        pltpu.sync_copy(rowsu_v, ugi_hbm.at[pl.ds(base, chunk)])
        cp_u2 = pltpu.async_copy(
            cat_hbm.at[idxu_v.at[pl.ds(chunk, chunk)]], rowsu_v, semu)
        cp_u2.wait()
        cp_i2.wait()
        merge()
        pltpu.sync_copy(rowsu_v, ugi_hbm.at[pl.ds(base + chunk, chunk)])

    return k(cat, uidx, iidx)


_COMB_BLK = 2048


def _combine_body(ugi, afT, w16, cafT, c0, o):
    ue = ugi[...][:, 0:_CAT]
    ie = ugi[...][:, _CAT:2 * _CAT]
    prod = ue * ie
    predT = _dg(w16[...], prod, _dn(0, 1))
    featT = _dg(cafT[...], afT[...], _dn(1, 0))
    o[...] = jax.nn.sigmoid(predT + featT + c0[...])


def _combine(ugi, afT, w16, cafT, c0):
    b = ugi.shape[0]
    c = _COMB_BLK
    grid = (b // c,)
    return pl.pallas_call(
        _combine_body,
        grid=grid,
        in_specs=[
            pl.BlockSpec((c, _TBL), lambda i: (i, 0)),
            pl.BlockSpec((8, c), lambda i: (0, i)),
            pl.BlockSpec((_CAT, 1), lambda i: (0, 0)),
            pl.BlockSpec((1, 8), lambda i: (0, 0)),
            pl.BlockSpec((1, 1), lambda i: (0, 0)),
        ],
        out_specs=pl.BlockSpec((1, c), lambda i: (0, i)),
        out_shape=jax.ShapeDtypeStruct((1, b), _F32),
    )(ugi, afT, w16, cafT, c0)


def kernel(user_indices, item_indices, anime_features, user_MF, item_MF,
           user_MLP, item_MLP, W_feat, b_feat, W1, b1, W2, b2, W3, b3, W4,
           b4, Wp, bp):
    cuT, ciT, cafT, w16, c0 = _collapse(
        W1, W2, W3.T, W4.T, Wp.T, W_feat,
        b_feat.reshape(1, -1), b1.reshape(1, -1), b2.reshape(1, -1),
        b3.reshape(1, -1), b4.reshape(1, -1), bp.reshape(1, 1))
    cat = _project(
        user_MLP.T, item_MLP.T, user_MF.T, item_MF.T, cuT, ciT)
    ugi = _sc_gather(cat, user_indices, item_indices)
    outT = _combine(ugi, anime_features.T, w16, cafT, c0)
    return outT.T


# speedup vs baseline: 7.0739x; 1.0069x over previous
import functools

import jax
import jax.numpy as jnp
from jax import lax
from jax.experimental import pallas as pl
from jax.experimental.pallas import tpu as pltpu
from jax.experimental.pallas import tpu_sc as plsc

_NC, _NS = 2, 16
_NW = _NC * _NS
_CAT = 16
_TBL = 128
_PACK = _TBL // _CAT
_F32 = jnp.float32


def _dn(a, b):
    return (((a,), (b,)), ((), ()))


def _dg(a, b, dn):
    return lax.dot_general(a, b, dn, preferred_element_type=_F32)


_COLS_BLK = 6144


def _project_body(u_mlpT, i_mlpT, u_mfT, i_mfT,
                  W1, W2, W3T, W4T, WpT, W_feat,
                  b_feat, b1, b2, b3, b4, bp,
                  cat_o, cafT_o, w16_o, c0_o, cu_s, ci_s):
    c = u_mlpT.shape[1]

    @pl.when(pl.program_id(0) == 0)
    def _():
        WpT_v = WpT[...]
        v4T = WpT_v[:, 8:40]
        u3T = _dg(v4T, W4T[...], _dn(1, 0))
        u2T = _dg(u3T, W3T[...], _dn(1, 0))
        u1T = _dg(u2T, W2[...], _dn(1, 1))
        cT = _dg(u1T, W1[...], _dn(1, 1))
        c3T = cT[:, 384:576]
        cu_s[...] = cT[:, 0:192]
        ci_s[...] = cT[:, 192:384]
        cafT_o[...] = _dg(c3T, W_feat[...], _dn(1, 1))
        row = lax.broadcasted_iota(jnp.int32, (_CAT, 40), 0)
        col = lax.broadcasted_iota(jnp.int32, (_CAT, 40), 1)
        sel = jnp.where((row == col) & (row < 8), 1.0, 0.0)
        w8 = _dg(sel, WpT_v, _dn(1, 1))
        r1 = lax.broadcasted_iota(jnp.int32, (_CAT, 1), 0)
        w16_o[...] = w8 + jnp.where((r1 >= 8) & (r1 < 10), 1.0, 0.0)
        c0_o[...] = (_dg(b_feat[...], c3T, _dn(1, 1))
                     + _dg(b1[...], u1T, _dn(1, 1))
                     + _dg(b2[...], u2T, _dn(1, 1))
                     + _dg(b3[...], u3T, _dn(1, 1))
                     + _dg(b4[...], v4T, _dn(1, 1))
                     + bp[...])

    pu = _dg(u_mlpT[...], cu_s[...], _dn(0, 1))
    pi = _dg(i_mlpT[...], ci_s[...], _dn(0, 1))
    r8 = lax.broadcasted_iota(jnp.int32, (8, 8), 0)
    c8 = lax.broadcasted_iota(jnp.int32, (8, 8), 1)
    eye8 = jnp.where(r8 == c8, 1.0, 0.0)
    umf = _dg(u_mfT[...], eye8, _dn(0, 0))
    imf = _dg(i_mfT[...], eye8, _dn(0, 0))
    ones = jnp.ones((c, 1), _F32)
    z6 = jnp.zeros((c, 6), _F32)
    ztail = jnp.zeros((c, _TBL - 2 * _CAT), _F32)
    cat_o[...] = jnp.concatenate(
        [umf, pu, ones, z6, imf, ones, pi, z6, ztail], axis=1)


def _project(user_MLPT, item_MLPT, user_MFT, item_MFT,
             W1, W2, W3T, W4T, WpT, W_feat, b_feat, b1, b2, b3, b4, bp):
    d, n = user_MLPT.shape
    c = _COLS_BLK
    grid = (pl.cdiv(n, c),)
    full = lambda a, b: pl.BlockSpec((a, b), lambda i: (0, 0))
    out_shape = (
        jax.ShapeDtypeStruct((n, _TBL), _F32),
        jax.ShapeDtypeStruct((1, 8), _F32),
        jax.ShapeDtypeStruct((_CAT, 1), _F32),
        jax.ShapeDtypeStruct((1, 1), _F32),
    )
    return pl.pallas_call(
        _project_body,
        grid=grid,
        compiler_params=pltpu.CompilerParams(
            fuse_transposed_lhs_in_matmul=True),
        in_specs=[
            pl.BlockSpec((d, c), lambda i: (0, i)),
            pl.BlockSpec((d, c), lambda i: (0, i)),
            pl.BlockSpec((8, c), lambda i: (0, i)),
            pl.BlockSpec((8, c), lambda i: (0, i)),
            full(576, 256), full(256, 128), full(64, 128), full(32, 64),
            full(1, 40), full(8, 192), full(1, 192), full(1, 256),
            full(1, 128), full(1, 64), full(1, 32), full(1, 1),
        ],
        out_specs=[
            pl.BlockSpec((c, _TBL), lambda i: (i, 0)),
            full(1, 8), full(_CAT, 1), full(1, 1),
        ],
        out_shape=out_shape,
        scratch_shapes=[
            pltpu.VMEM((1, 192), _F32),
            pltpu.VMEM((1, 192), _F32),
        ],
    )(user_MLPT, item_MLPT, user_MFT, item_MFT,
      W1, W2, W3T, W4T, WpT, W_feat, b_feat, b1, b2, b3, b4, bp)


def _sc_gather(cat, uidx, iidx):
    b = uidx.shape[0]
    bpw = b // _NW
    chunk = bpw // 2
    mesh = plsc.VectorSubcoreMesh(core_axis_name="c", subcore_axis_name="s")
    out_type = jax.ShapeDtypeStruct((b, _TBL), _F32)

    @functools.partial(
        pl.kernel, mesh=mesh, out_type=out_type,
        scratch_types=[
            pltpu.VMEM((bpw,), jnp.int32),
            pltpu.VMEM((bpw,), jnp.int32),
            pltpu.VMEM((chunk, _TBL), _F32),
            pltpu.VMEM((chunk, _TBL), _F32),
            pltpu.SemaphoreType.DMA,
            pltpu.SemaphoreType.DMA,
        ],
    )
    def k(cat_hbm, uidx_hbm, iidx_hbm, ugi_hbm,
          idxu_v, idxi_v, rowsu_v, rowsi_v, semu, semi):
        wid = lax.axis_index("s") * _NC + lax.axis_index("c")
        base = wid * bpw
        pltpu.sync_copy(uidx_hbm.at[pl.ds(base, bpw)], idxu_v)
        pltpu.sync_copy(iidx_hbm.at[pl.ds(base, bpw)], idxi_v)
        cp_u = pltpu.async_copy(
            cat_hbm.at[idxu_v.at[pl.ds(0, chunk)]], rowsu_v, semu)
        cp_i = pltpu.async_copy(
            cat_hbm.at[idxi_v.at[pl.ds(0, chunk)]], rowsi_v, semi)
        def merge():
            @pl.loop(0, chunk, step=8)
            def _(r):
                for j in range(8):
                    sl = pl.ds(_CAT, _CAT)
                    rowsu_v[r + j, sl] = rowsi_v[r + j, sl]

        cp_u.wait()
        cp_i.wait()
        merge()
        cp_i2 = pltpu.async_copy(
            cat_hbm.at[idxi_v.at[pl.ds(chunk, chunk)]], rowsi_v, semi)
        pltpu.sync_copy(rowsu_v, ugi_hbm.at[pl.ds(base, chunk)])
        cp_u2 = pltpu.async_copy(
            cat_hbm.at[idxu_v.at[pl.ds(chunk, chunk)]], rowsu_v, semu)
        cp_u2.wait()
        cp_i2.wait()
        merge()
        pltpu.sync_copy(rowsu_v, ugi_hbm.at[pl.ds(base + chunk, chunk)])

    return k(cat, uidx, iidx)


_COMB_BLK = 2048


def _combine_body(ugi, afT, w16, cafT, c0, o):
    ue = ugi[...][:, 0:_CAT]
    ie = ugi[...][:, _CAT:2 * _CAT]
    prod = ue * ie
    predT = _dg(w16[...], prod, _dn(0, 1))
    featT = _dg(cafT[...], afT[...], _dn(1, 0))
    o[...] = jax.nn.sigmoid(predT + featT + c0[...])


def _combine(ugi, afT, w16, cafT, c0):
    b = ugi.shape[0]
    c = _COMB_BLK
    grid = (b // c,)
    return pl.pallas_call(
        _combine_body,
        grid=grid,
        in_specs=[
            pl.BlockSpec((c, _TBL), lambda i: (i, 0)),
            pl.BlockSpec((8, c), lambda i: (0, i)),
            pl.BlockSpec((_CAT, 1), lambda i: (0, 0)),
            pl.BlockSpec((1, 8), lambda i: (0, 0)),
            pl.BlockSpec((1, 1), lambda i: (0, 0)),
        ],
        out_specs=pl.BlockSpec((1, c), lambda i: (0, i)),
        out_shape=jax.ShapeDtypeStruct((1, b), _F32),
    )(ugi, afT, w16, cafT, c0)


def kernel(user_indices, item_indices, anime_features, user_MF, item_MF,
           user_MLP, item_MLP, W_feat, b_feat, W1, b1, W2, b2, W3, b3, W4,
           b4, Wp, bp):
    cat, cafT, w16, c0 = _project(
        user_MLP.T, item_MLP.T, user_MF.T, item_MF.T,
        W1, W2, W3.T, W4.T, Wp.T, W_feat,
        b_feat.reshape(1, -1), b1.reshape(1, -1), b2.reshape(1, -1),
        b3.reshape(1, -1), b4.reshape(1, -1), bp.reshape(1, 1))
    ugi = _sc_gather(cat, user_indices, item_indices)
    outT = _combine(ugi, anime_features.T, w16, cafT, c0)
    return outT.T


# speedup vs baseline: 7.1896x; 1.0163x over previous
import functools

import jax
import jax.numpy as jnp
from jax import lax
from jax.experimental import pallas as pl
from jax.experimental.pallas import tpu as pltpu
from jax.experimental.pallas import tpu_sc as plsc

_NC, _NS = 2, 16
_NW = _NC * _NS
_CAT = 16
_TBL = 128
_PACK = _TBL // _CAT
_F32 = jnp.float32


def _dn(a, b):
    return (((a,), (b,)), ((), ()))


def _dg(a, b, dn):
    return lax.dot_general(a, b, dn, preferred_element_type=_F32)


_COLS_BLK = 6144


def _project_body(u_mlpT, i_mlpT, u_mfT, i_mfT,
                  W1, W2, W3T, W4T, WpT, W_feat,
                  b_feat, b1, b2, b3, b4, bp,
                  cat_o, cafT_o, w16_o, c0_o, cu_s, ci_s):
    c = u_mlpT.shape[1]

    @pl.when(pl.program_id(0) == 0)
    def _():
        WpT_v = WpT[...]
        v4T = WpT_v[:, 8:40]
        u3T = _dg(v4T, W4T[...], _dn(1, 0))
        u2T = _dg(u3T, W3T[...], _dn(1, 0))
        u1T = _dg(u2T, W2[...], _dn(1, 1))
        cT = _dg(u1T, W1[...], _dn(1, 1))
        c3T = cT[:, 384:576]
        cu_s[...] = cT[:, 0:192]
        ci_s[...] = cT[:, 192:384]
        cafT_o[...] = _dg(c3T, W_feat[...], _dn(1, 1))
        row = lax.broadcasted_iota(jnp.int32, (_CAT, 40), 0)
        col = lax.broadcasted_iota(jnp.int32, (_CAT, 40), 1)
        sel = jnp.where((row == col) & (row < 8), 1.0, 0.0)
        w8 = _dg(sel, WpT_v, _dn(1, 1))
        r1 = lax.broadcasted_iota(jnp.int32, (_CAT, 1), 0)
        w16_o[...] = w8 + jnp.where((r1 >= 8) & (r1 < 10), 1.0, 0.0)
        c0_o[...] = (_dg(b_feat[...], c3T, _dn(1, 1))
                     + _dg(b1[...], u1T, _dn(1, 1))
                     + _dg(b2[...], u2T, _dn(1, 1))
                     + _dg(b3[...], u3T, _dn(1, 1))
                     + _dg(b4[...], v4T, _dn(1, 1))
                     + bp[...])

    pu = _dg(u_mlpT[...], cu_s[...], _dn(0, 1))
    pi = _dg(i_mlpT[...], ci_s[...], _dn(0, 1))
    r8 = lax.broadcasted_iota(jnp.int32, (8, 8), 0)
    c8 = lax.broadcasted_iota(jnp.int32, (8, 8), 1)
    eye8 = jnp.where(r8 == c8, 1.0, 0.0)
    umf = _dg(u_mfT[...], eye8, _dn(0, 0))
    imf = _dg(i_mfT[...], eye8, _dn(0, 0))
    ones = jnp.ones((c, 1), _F32)
    z6 = jnp.zeros((c, 6), _F32)
    ztail = jnp.zeros((c, _TBL - 2 * _CAT), _F32)
    cat_o[...] = jnp.concatenate(
        [umf, pu, ones, z6, imf, ones, pi, z6, ztail], axis=1)


def _project(user_MLPT, item_MLPT, user_MFT, item_MFT,
             W1, W2, W3T, W4T, WpT, W_feat, b_feat, b1, b2, b3, b4, bp):
    d, n = user_MLPT.shape
    c = _COLS_BLK
    grid = (pl.cdiv(n, c),)
    full = lambda a, b: pl.BlockSpec((a, b), lambda i: (0, 0))
    out_shape = (
        jax.ShapeDtypeStruct((n, _TBL), _F32),
        jax.ShapeDtypeStruct((1, 8), _F32),
        jax.ShapeDtypeStruct((_CAT, 1), _F32),
        jax.ShapeDtypeStruct((1, 1), _F32),
    )
    return pl.pallas_call(
        _project_body,
        grid=grid,
        compiler_params=pltpu.CompilerParams(
            fuse_transposed_lhs_in_matmul=True),
        in_specs=[
            pl.BlockSpec((d, c), lambda i: (0, i)),
            pl.BlockSpec((d, c), lambda i: (0, i)),
            pl.BlockSpec((8, c), lambda i: (0, i)),
            pl.BlockSpec((8, c), lambda i: (0, i)),
            full(576, 256), full(256, 128), full(64, 128), full(32, 64),
            full(1, 40), full(8, 192), full(1, 192), full(1, 256),
            full(1, 128), full(1, 64), full(1, 32), full(1, 1),
        ],
        out_specs=[
            pl.BlockSpec((c, _TBL), lambda i: (i, 0)),
            full(1, 8), full(_CAT, 1), full(1, 1),
        ],
        out_shape=out_shape,
        scratch_shapes=[
            pltpu.VMEM((1, 192), _F32),
            pltpu.VMEM((1, 192), _F32),
        ],
    )(user_MLPT, item_MLPT, user_MFT, item_MFT,
      W1, W2, W3T, W4T, WpT, W_feat, b_feat, b1, b2, b3, b4, bp)


def _sc_gather(cat, uidx, iidx):
    b = uidx.shape[0]
    bpw = b // _NW
    chunk = bpw // 2
    mesh = plsc.VectorSubcoreMesh(core_axis_name="c", subcore_axis_name="s")
    out_type = jax.ShapeDtypeStruct((b, _TBL), _F32)

    @functools.partial(
        pl.kernel, mesh=mesh, out_type=out_type,
        scratch_types=[
            pltpu.VMEM((bpw,), jnp.int32),
            pltpu.VMEM((bpw,), jnp.int32),
            pltpu.VMEM((chunk, _TBL), _F32),
            pltpu.VMEM((chunk, _TBL), _F32),
            pltpu.SemaphoreType.DMA,
            pltpu.SemaphoreType.DMA,
        ],
    )
    def k(cat_hbm, uidx_hbm, iidx_hbm, ugi_hbm,
          idxu_v, idxi_v, rowsu_v, rowsi_v, semu, semi):
        wid = lax.axis_index("s") * _NC + lax.axis_index("c")
        base = wid * bpw
        pltpu.sync_copy(uidx_hbm.at[pl.ds(base, bpw)], idxu_v)
        pltpu.sync_copy(iidx_hbm.at[pl.ds(base, bpw)], idxi_v)
        cp_u = pltpu.async_copy(
            cat_hbm.at[idxu_v.at[pl.ds(0, chunk)]], rowsu_v, semu)
        cp_i = pltpu.async_copy(
            cat_hbm.at[idxi_v.at[pl.ds(0, chunk)]], rowsi_v, semi)
        def merge():
            @pl.loop(0, chunk, step=8)
            def _(r):
                for j in range(8):
                    sl = pl.ds(_CAT, _CAT)
                    rowsu_v[r + j, sl] = rowsi_v[r + j, sl]

        cp_u.wait()
        cp_i.wait()
        merge()
        cp_i2 = pltpu.async_copy(
            cat_hbm.at[idxi_v.at[pl.ds(chunk, chunk)]], rowsi_v, semi)
        pltpu.sync_copy(rowsu_v, ugi_hbm.at[pl.ds(base, chunk)])
        cp_u2 = pltpu.async_copy(
            cat_hbm.at[idxu_v.at[pl.ds(chunk, chunk)]], rowsu_v, semu)
        cp_u2.wait()
        cp_i2.wait()
        merge()
        pltpu.sync_copy(rowsu_v, ugi_hbm.at[pl.ds(base + chunk, chunk)])

    return k(cat, uidx, iidx)


_COMB_BLK = 16384


def _combine_body(ugi, afT, w16, cafT, c0, o):
    ue = ugi[...][:, 0:_CAT]
    ie = ugi[...][:, _CAT:2 * _CAT]
    prod = ue * ie
    predT = _dg(w16[...], prod, _dn(0, 1))
    featT = _dg(cafT[...], afT[...], _dn(1, 0))
    o[...] = jax.nn.sigmoid(predT + featT + c0[...])


def _combine(ugi, afT, w16, cafT, c0):
    b = ugi.shape[0]
    c = _COMB_BLK
    grid = (b // c,)
    return pl.pallas_call(
        _combine_body,
        grid=grid,
        in_specs=[
            pl.BlockSpec((c, _TBL), lambda i: (i, 0)),
            pl.BlockSpec((8, c), lambda i: (0, i)),
            pl.BlockSpec((_CAT, 1), lambda i: (0, 0)),
            pl.BlockSpec((1, 8), lambda i: (0, 0)),
            pl.BlockSpec((1, 1), lambda i: (0, 0)),
        ],
        out_specs=pl.BlockSpec((1, c), lambda i: (0, i)),
        out_shape=jax.ShapeDtypeStruct((1, b), _F32),
    )(ugi, afT, w16, cafT, c0)


def kernel(user_indices, item_indices, anime_features, user_MF, item_MF,
           user_MLP, item_MLP, W_feat, b_feat, W1, b1, W2, b2, W3, b3, W4,
           b4, Wp, bp):
    cat, cafT, w16, c0 = _project(
        user_MLP.T, item_MLP.T, user_MF.T, item_MF.T,
        W1, W2, W3.T, W4.T, Wp.T, W_feat,
        b_feat.reshape(1, -1), b1.reshape(1, -1), b2.reshape(1, -1),
        b3.reshape(1, -1), b4.reshape(1, -1), bp.reshape(1, 1))
    ugi = _sc_gather(cat, user_indices, item_indices)
    outT = _combine(ugi, anime_features.T, w16, cafT, c0)
    return outT.T
